# baseline (speedup 1.0000x reference)
import jax
import jax.numpy as jnp
from jax import lax
from jax.experimental import pallas as pl
from jax.experimental.pallas import tpu as pltpu
from jax.experimental.pallas import tpu_sc as plsc

N = 320000
D = 128
S = 10000
NC = 2
NT = 16
BS = 40
H = S // NC
B = 128
RPT = N // NT
NB = RPT // B
NBUF = 4
PF = 3
TAIL_BASE = NT * NB * B
TAIL_BATCHES = (N - TAIL_BASE) // B
ACC_ROWS = 5120
ZPT = ACC_ROWS // NT
WCHUNKS = H // BS
DIV_M = (1 << 21) // BS + 1


def _div_bs(x):
    return (x * DIV_M) >> 21


def _seg_sum_body(data_hbm, ids_hbm, out_hbm, ids_buf, ids_x, datas, idxs,
                  klist, acc, sems_in, sems_s):
    cid = lax.axis_index("c")
    tid = lax.axis_index("s")

    def zrow(i, _):
        def zcol(j, _):
            datas[0][i, pl.ds(j * 16, 16)] = jnp.zeros((16,), jnp.float32)
            return 0
        return lax.fori_loop(0, D // 16, zcol, 0)
    lax.fori_loop(0, B, zrow, 0)
    z0 = tid * ZPT
    pltpu.sync_copy(datas[0], acc.at[pl.ds(z0, B)])
    pltpu.sync_copy(datas[0], acc.at[pl.ds(z0 + B, B)])
    pltpu.sync_copy(datas[0].at[pl.ds(0, ZPT - 2 * B)],
                    acc.at[pl.ds(z0 + 2 * B, ZPT - 2 * B)])

    pltpu.sync_copy(ids_hbm.at[pl.ds(tid * NB * B, NB * B)], ids_buf)

    plsc.subcore_barrier()

    def gen_idx(ids_ref, o, idx_b):
        def cchunk(j, _):
            ids16 = ids_ref[pl.ds(o + j * 16, 16)]
            blk = _div_bs(ids16)
            rel = (blk >> 1) * BS + (ids16 - blk * BS)
            own = (blk & 1) == cid
            idx_b[pl.ds(j * 16, 16)] = jnp.where(own, rel, H)
            return 0
        lax.fori_loop(0, B // 16, cchunk, 0)

    def owned_cond(ids_ref, o):
        first = ids_ref[pl.ds(o, 16)][0]
        last = ids_ref[pl.ds(o + B - 16, 16)][15]
        fb = _div_bs(first)
        lb = _div_bs(last)
        return (lb > fb) | ((fb & 1) == cid)

    def scan_chunk(k, m):
        c = owned_cond(ids_buf, k * B)

        @pl.when(c)
        def _():
            klist[m] = k
        return jnp.where(c, m + 1, m)
    M = lax.fori_loop(0, NB, scan_chunk, jnp.int32(0))

    def issue_in(i, b):
        k = klist[i]
        pltpu.async_copy(data_hbm.at[pl.ds((tid * NB + k) * B, B)],
                         datas[b], sems_in[b])

    def drain_scatter(b):
        pltpu.make_async_copy(datas[b], acc.at[idxs[b]], sems_s[b]).wait()

    for b in range(PF):
        @pl.when(b < M)
        def _(b=b):
            issue_in(b, b)

    def group(g, _):
        for b in range(NBUF):
            i = g * NBUF + b
            pre = i + PF
            bp = (b + PF) % NBUF

            @pl.when(pre < M)
            def _(pre=pre, bp=bp):
                @pl.when(pre >= NBUF)
                def _():
                    drain_scatter(bp)
                issue_in(pre, bp)

            @pl.when(i < M)
            def _(i=i, b=b):
                pltpu.make_async_copy(
                    data_hbm.at[pl.ds(0, B)], datas[b], sems_in[b]).wait()
                gen_idx(ids_buf, klist[i] * B, idxs[b])
                pltpu.async_copy(datas[b], acc.at[idxs[b]], sems_s[b],
                                 add=True)
        return 0
    lax.fori_loop(0, -(-NB // NBUF), group, 0)

    for b in range(NBUF):
        @pl.when(b < M)
        def _(b=b):
            drain_scatter(b)

    @pl.when(tid < TAIL_BATCHES)
    def _():
        row0 = TAIL_BASE + tid * B
        pltpu.sync_copy(ids_hbm.at[pl.ds(row0, B)], ids_x)

        @pl.when(owned_cond(ids_x, 0))
        def _():
            pltpu.sync_copy(data_hbm.at[pl.ds(row0, B)], datas[0])
            gen_idx(ids_x, 0, idxs[0])
            pltpu.sync_copy(datas[0], acc.at[idxs[0]], add=True)

    plsc.subcore_barrier()

    def wout(c, _):
        chunk = tid + NT * c

        @pl.when(chunk < WCHUNKS)
        def _():
            pltpu.sync_copy(acc.at[pl.ds(chunk * BS, BS)],
                            out_hbm.at[pl.ds((2 * chunk + cid) * BS, BS)])
        return 0
    lax.fori_loop(0, -(-WCHUNKS // NT), wout, 0)


_seg_sum = pl.kernel(
    _seg_sum_body,
    out_type=jax.ShapeDtypeStruct((S, D), jnp.float32),
    mesh=plsc.VectorSubcoreMesh(core_axis_name="c", subcore_axis_name="s"),
    scratch_types=[
        pltpu.VMEM((NB * B,), jnp.int32),
        pltpu.VMEM((B,), jnp.int32),
        [pltpu.VMEM((B, D), jnp.float32) for _ in range(NBUF)],
        [pltpu.VMEM((B,), jnp.int32) for _ in range(NBUF)],
        pltpu.SMEM((NB,), jnp.int32),
        pltpu.VMEM_SHARED((ACC_ROWS, D), jnp.float32),
        [pltpu.SemaphoreType.DMA for _ in range(NBUF)],
        [pltpu.SemaphoreType.DMA for _ in range(NBUF)],
    ],
)


def kernel(data, segment_ids):
    return _seg_sum(data, segment_ids)


# speedup vs baseline: 1.0348x; 1.0348x over previous
import jax
import jax.numpy as jnp
from jax import lax
from jax.experimental import pallas as pl
from jax.experimental.pallas import tpu as pltpu
from jax.experimental.pallas import tpu_sc as plsc

N = 320000
D = 128
S = 10000
NC = 2
NT = 16
BS = 40
H = S // NC
B = 128
RPT = N // NT
NB = RPT // B
NBUF = 4
PF = 2
TAIL_BASE = NT * NB * B
TAIL_BATCHES = (N - TAIL_BASE) // B
ACC_ROWS = 5120
ZPT = ACC_ROWS // NT
WCHUNKS = H // BS
DIV_M = (1 << 21) // BS + 1


def _div_bs(x):
    return (x * DIV_M) >> 21


def _seg_sum_body(data_hbm, ids_hbm, out_hbm, ids_buf, ids_x, datas, idxs,
                  klist, acc, sems_in, sems_s):
    cid = lax.axis_index("c")
    tid = lax.axis_index("s")

    ids_desc = pltpu.async_copy(ids_hbm.at[pl.ds(tid * NB * B, NB * B)],
                                ids_buf, sems_in[0])

    def zrow(i, _):
        def zcol(j, _):
            datas[0][i, pl.ds(j * 16, 16)] = jnp.zeros((16,), jnp.float32)
            return 0
        return lax.fori_loop(0, D // 16, zcol, 0)
    lax.fori_loop(0, B, zrow, 0)
    z0 = tid * ZPT
    pltpu.sync_copy(datas[0], acc.at[pl.ds(z0, B)])
    pltpu.sync_copy(datas[0], acc.at[pl.ds(z0 + B, B)])
    pltpu.sync_copy(datas[0].at[pl.ds(0, ZPT - 2 * B)],
                    acc.at[pl.ds(z0 + 2 * B, ZPT - 2 * B)])

    ids_desc.wait()

    plsc.subcore_barrier()

    def gen_idx(ids_ref, o, idx_b):
        def cchunk(j, _):
            ids16 = ids_ref[pl.ds(o + j * 16, 16)]
            blk = _div_bs(ids16)
            rel = (blk >> 1) * BS + (ids16 - blk * BS)
            own = (blk & 1) == cid
            idx_b[pl.ds(j * 16, 16)] = jnp.where(own, rel, H)
            return 0
        lax.fori_loop(0, B // 16, cchunk, 0)

    def owned_cond(ids_ref, o):
        first = ids_ref[pl.ds(o, 16)][0]
        last = ids_ref[pl.ds(o + B - 16, 16)][15]
        fb = _div_bs(first)
        lb = _div_bs(last)
        return (lb > fb) | ((fb & 1) == cid)

    def scan_chunk(k, m):
        c = owned_cond(ids_buf, k * B)

        @pl.when(c)
        def _():
            klist[m] = k
        return jnp.where(c, m + 1, m)
    M = lax.fori_loop(0, NB, scan_chunk, jnp.int32(0))

    def issue_in(i, b):
        k = klist[i]
        pltpu.async_copy(data_hbm.at[pl.ds((tid * NB + k) * B, B)],
                         datas[b], sems_in[b])

    def drain_scatter(b):
        pltpu.make_async_copy(datas[b], acc.at[idxs[b]], sems_s[b]).wait()

    for b in range(PF):
        @pl.when(b < M)
        def _(b=b):
            issue_in(b, b)

    def group(g, _):
        for b in range(NBUF):
            i = g * NBUF + b
            pre = i + PF
            bp = (b + PF) % NBUF

            @pl.when(pre < M)
            def _(pre=pre, bp=bp):
                @pl.when(pre >= NBUF)
                def _():
                    drain_scatter(bp)
                issue_in(pre, bp)

            @pl.when(i < M)
            def _(i=i, b=b):
                pltpu.make_async_copy(
                    data_hbm.at[pl.ds(0, B)], datas[b], sems_in[b]).wait()
                gen_idx(ids_buf, klist[i] * B, idxs[b])
                pltpu.async_copy(datas[b], acc.at[idxs[b]], sems_s[b],
                                 add=True)
        return 0
    lax.fori_loop(0, -(-NB // NBUF), group, 0)

    for b in range(NBUF):
        @pl.when(b < M)
        def _(b=b):
            drain_scatter(b)

    @pl.when(tid < TAIL_BATCHES)
    def _():
        row0 = TAIL_BASE + tid * B
        pltpu.sync_copy(ids_hbm.at[pl.ds(row0, B)], ids_x)

        @pl.when(owned_cond(ids_x, 0))
        def _():
            pltpu.sync_copy(data_hbm.at[pl.ds(row0, B)], datas[0])
            gen_idx(ids_x, 0, idxs[0])
            pltpu.sync_copy(datas[0], acc.at[idxs[0]], add=True)

    plsc.subcore_barrier()

    def wout(c, _):
        chunk = tid + NT * c

        @pl.when(chunk < WCHUNKS)
        def _():
            pltpu.sync_copy(acc.at[pl.ds(chunk * BS, BS)],
                            out_hbm.at[pl.ds((2 * chunk + cid) * BS, BS)])
        return 0
    lax.fori_loop(0, -(-WCHUNKS // NT), wout, 0)


_seg_sum = pl.kernel(
    _seg_sum_body,
    out_type=jax.ShapeDtypeStruct((S, D), jnp.float32),
    mesh=plsc.VectorSubcoreMesh(core_axis_name="c", subcore_axis_name="s"),
    scratch_types=[
        pltpu.VMEM((NB * B,), jnp.int32),
        pltpu.VMEM((B,), jnp.int32),
        [pltpu.VMEM((B, D), jnp.float32) for _ in range(NBUF)],
        [pltpu.VMEM((B,), jnp.int32) for _ in range(NBUF)],
        pltpu.SMEM((NB,), jnp.int32),
        pltpu.VMEM_SHARED((ACC_ROWS, D), jnp.float32),
        [pltpu.SemaphoreType.DMA for _ in range(NBUF)],
        [pltpu.SemaphoreType.DMA for _ in range(NBUF)],
    ],
)


def kernel(data, segment_ids):
    return _seg_sum(data, segment_ids)


# speedup vs baseline: 1.0407x; 1.0056x over previous
import jax
import jax.numpy as jnp
from jax import lax
from jax.experimental import pallas as pl
from jax.experimental.pallas import tpu as pltpu
from jax.experimental.pallas import tpu_sc as plsc

N = 320000
D = 128
S = 10000
NC = 2
NT = 16
BS = 40
H = S // NC
B = 128
RPT = N // NT
NB = RPT // B
NBUF = 4
PF = 2
TAIL_BASE = NT * NB * B
TAIL_BATCHES = (N - TAIL_BASE) // B
ACC_ROWS = 5120
ZPT = ACC_ROWS // NT
WCHUNKS = H // BS
DIV_M = (1 << 21) // BS + 1


def _div_bs(x):
    return (x * DIV_M) >> 21


def _seg_sum_body(data_hbm, ids_hbm, out_hbm, ids_buf, ids_x, datas, idxs,
                  klist, acc, sems_in, sems_s):
    cid = lax.axis_index("c")
    tid = lax.axis_index("s")

    ids_desc = pltpu.async_copy(ids_hbm.at[pl.ds(tid * NB * B, NB * B)],
                                ids_buf, sems_in[0])

    def zrow(i, _):
        def zcol(j, _):
            datas[0][i, pl.ds(j * 16, 16)] = jnp.zeros((16,), jnp.float32)
            return 0
        return lax.fori_loop(0, D // 16, zcol, 0)
    lax.fori_loop(0, B, zrow, 0)
    z0 = tid * ZPT
    zd0 = pltpu.async_copy(datas[0], acc.at[pl.ds(z0, B)], sems_s[0])
    zd1 = pltpu.async_copy(datas[0], acc.at[pl.ds(z0 + B, B)], sems_s[0])
    zd2 = pltpu.async_copy(datas[0].at[pl.ds(0, ZPT - 2 * B)],
                           acc.at[pl.ds(z0 + 2 * B, ZPT - 2 * B)], sems_s[0])

    ids_desc.wait()

    def gen_idx(ids_ref, o, idx_b):
        def cchunk(j, _):
            ids16 = ids_ref[pl.ds(o + j * 16, 16)]
            blk = _div_bs(ids16)
            rel = (blk >> 1) * BS + (ids16 - blk * BS)
            own = (blk & 1) == cid
            idx_b[pl.ds(j * 16, 16)] = jnp.where(own, rel, H)
            return 0
        lax.fori_loop(0, B // 16, cchunk, 0)

    def owned_cond(ids_ref, o):
        first = ids_ref[pl.ds(o, 16)][0]
        last = ids_ref[pl.ds(o + B - 16, 16)][15]
        fb = _div_bs(first)
        lb = _div_bs(last)
        return (lb > fb) | ((fb & 1) == cid)

    def scan_chunk(k, m):
        c = owned_cond(ids_buf, k * B)

        @pl.when(c)
        def _():
            klist[m] = k
        return jnp.where(c, m + 1, m)
    M = lax.fori_loop(0, NB, scan_chunk, jnp.int32(0))

    zd0.wait()
    zd1.wait()
    zd2.wait()
    plsc.subcore_barrier()

    def issue_in(i, b):
        k = klist[i]
        pltpu.async_copy(data_hbm.at[pl.ds((tid * NB + k) * B, B)],
                         datas[b], sems_in[b])

    def drain_scatter(b):
        pltpu.make_async_copy(datas[b], acc.at[idxs[b]], sems_s[b]).wait()

    for b in range(PF):
        @pl.when(b < M)
        def _(b=b):
            issue_in(b, b)

    def group(g, _):
        for b in range(NBUF):
            i = g * NBUF + b
            pre = i + PF
            bp = (b + PF) % NBUF

            @pl.when(pre < M)
            def _(pre=pre, bp=bp):
                @pl.when(pre >= NBUF)
                def _():
                    drain_scatter(bp)
                issue_in(pre, bp)

            @pl.when(i < M)
            def _(i=i, b=b):
                pltpu.make_async_copy(
                    data_hbm.at[pl.ds(0, B)], datas[b], sems_in[b]).wait()
                gen_idx(ids_buf, klist[i] * B, idxs[b])
                pltpu.async_copy(datas[b], acc.at[idxs[b]], sems_s[b],
                                 add=True)
        return 0
    lax.fori_loop(0, -(-NB // NBUF), group, 0)

    for b in range(NBUF):
        @pl.when(b < M)
        def _(b=b):
            drain_scatter(b)

    @pl.when(tid < TAIL_BATCHES)
    def _():
        row0 = TAIL_BASE + tid * B
        pltpu.sync_copy(ids_hbm.at[pl.ds(row0, B)], ids_x)

        @pl.when(owned_cond(ids_x, 0))
        def _():
            pltpu.sync_copy(data_hbm.at[pl.ds(row0, B)], datas[0])
            gen_idx(ids_x, 0, idxs[0])
            pltpu.sync_copy(datas[0], acc.at[idxs[0]], add=True)

    plsc.subcore_barrier()

    def wout(c, _):
        chunk = tid + NT * c

        @pl.when(chunk < WCHUNKS)
        def _():
            pltpu.sync_copy(acc.at[pl.ds(chunk * BS, BS)],
                            out_hbm.at[pl.ds((2 * chunk + cid) * BS, BS)])
        return 0
    lax.fori_loop(0, -(-WCHUNKS // NT), wout, 0)


_seg_sum = pl.kernel(
    _seg_sum_body,
    out_type=jax.ShapeDtypeStruct((S, D), jnp.float32),
    mesh=plsc.VectorSubcoreMesh(core_axis_name="c", subcore_axis_name="s"),
    scratch_types=[
        pltpu.VMEM((NB * B,), jnp.int32),
        pltpu.VMEM((B,), jnp.int32),
        [pltpu.VMEM((B, D), jnp.float32) for _ in range(NBUF)],
        [pltpu.VMEM((B,), jnp.int32) for _ in range(NBUF)],
        pltpu.SMEM((NB,), jnp.int32),
        pltpu.VMEM_SHARED((ACC_ROWS, D), jnp.float32),
        [pltpu.SemaphoreType.DMA for _ in range(NBUF)],
        [pltpu.SemaphoreType.DMA for _ in range(NBUF)],
    ],
)


def kernel(data, segment_ids):
    return _seg_sum(data, segment_ids)
